# baseline (speedup 1.0000x reference)
import functools

import jax
import jax.numpy as jnp
from jax import lax
from jax.experimental import pallas as pl
from jax.experimental.pallas import tpu as pltpu
from jax.experimental.pallas import tpu_sc as plsc

NUM_TASKS = 100000
EMBED_DIM = 128
BATCH = 16384

_INFO = plsc.get_sparse_core_info()
_NC = _INFO.num_cores
_NS = _INFO.num_subcores
_NW = _NC * _NS
_CHUNK = 128
_B_PER_W = BATCH // _NW
_N_ROWS = _B_PER_W // _CHUNK
_SPLITS = [(0, 0, 128), (1, 0, 128), (2, 0, 128), (3, 0, 96), (3, 96, 32)]


def _make_gather():
  mesh = plsc.VectorSubcoreMesh(core_axis_name="c", subcore_axis_name="s")

  @functools.partial(
      pl.kernel,
      mesh=mesh,
      out_type=jax.ShapeDtypeStruct((BATCH, EMBED_DIM), jnp.float32),
      scratch_types=[
          pltpu.VMEM((_N_ROWS, _CHUNK), jnp.int32),
          pltpu.VMEM((_B_PER_W, EMBED_DIM), jnp.float32),
          pltpu.SemaphoreType.DMA((len(_SPLITS),)),
          pltpu.SemaphoreType.DMA,
      ],
  )
  def gather_kernel(idx_hbm, table_hbm, out_hbm, idx_v, rows_v, gsem, osem):
    wid = lax.axis_index("s") * _NC + lax.axis_index("c")
    base = wid * _B_PER_W
    pltpu.sync_copy(idx_hbm.at[pl.ds(wid * _N_ROWS, 1)],
                    idx_v.at[pl.ds(0, 1)])
    gathers = [
        pltpu.async_copy(
            table_hbm.at[idx_v.at[0]],
            rows_v.at[pl.ds(0, _CHUNK)],
            gsem.at[0],
        )
    ]
    pltpu.sync_copy(idx_hbm.at[pl.ds(wid * _N_ROWS, _N_ROWS)], idx_v)
    off = _CHUNK
    offsets = [0]
    for j, (row, col, ln) in enumerate(_SPLITS[1:], start=1):
      gathers.append(
          pltpu.async_copy(
              table_hbm.at[idx_v.at[row, pl.ds(col, ln)]],
              rows_v.at[pl.ds(off, ln)],
              gsem.at[j],
          ))
      offsets.append(off)
      off += ln
    outs = []
    for j, (row, col, ln) in enumerate(_SPLITS):
      gathers[j].wait()
      outs.append(
          pltpu.async_copy(
              rows_v.at[pl.ds(offsets[j], ln)],
              out_hbm.at[pl.ds(base + offsets[j], ln)],
              osem,
          ))
    for o in outs:
      o.wait()

  return gather_kernel


_gather = _make_gather()


@jax.jit
def kernel(task_id, table):
  idx = task_id.astype(jnp.int32).reshape(BATCH // _CHUNK, _CHUNK)
  return _gather(idx, table)
